# baseline (speedup 1.0000x reference)
import functools

import jax
import jax.numpy as jnp
from jax import lax
from jax.experimental import pallas as pl
from jax.experimental.pallas import tpu as pltpu
from jax.experimental.pallas import tpu_sc as plsc

VOCAB = 1000000
DIM = 64
BATCH = 16384
HIST = 200

NC = 2
NS = 16
NW = NC * NS

RPW = BATCH // NW
CROWS = 4
NCHUNK = RPW // CROWS
NBUF = 2

_mesh = plsc.VectorSubcoreMesh(core_axis_name="c", subcore_axis_name="s")


@functools.partial(
    pl.kernel,
    out_type=jax.ShapeDtypeStruct((BATCH, HIST, DIM), jnp.float32),
    mesh=_mesh,
    scratch_types=[
        pltpu.VMEM((NBUF, CROWS, HIST), jnp.int32),
        pltpu.VMEM((NBUF, CROWS, HIST, DIM), jnp.float32),
        pltpu.SemaphoreType.DMA((NBUF,)),
        pltpu.SemaphoreType.DMA((NBUF,)),
        pltpu.SemaphoreType.DMA((NBUF,)),
    ],
    compiler_params=pltpu.CompilerParams(use_tc_tiling_on_sc=False),
)
def _embed(x_hbm, w_hbm, out_hbm, idx_v, rows_v, isem, gsem, ssem):
    wid = lax.axis_index("s") * NC + lax.axis_index("c")
    row0 = wid * RPW

    def fire_gathers(b):
        for k in range(CROWS):
            pltpu.async_copy(w_hbm.at[idx_v.at[b, k]], rows_v.at[b, k],
                             gsem.at[b])

    def wait_gathers(b):
        for k in range(CROWS):
            pltpu.make_async_copy(w_hbm.at[idx_v.at[b, k]], rows_v.at[b, k],
                                  gsem.at[b]).wait()

    for b in range(NBUF):
        pltpu.async_copy(x_hbm.at[pl.ds(row0 + b * CROWS, CROWS)],
                         idx_v.at[b], isem.at[b])
    for b in range(NBUF):
        pltpu.make_async_copy(x_hbm.at[pl.ds(row0 + b * CROWS, CROWS)],
                              idx_v.at[b], isem.at[b]).wait()
        fire_gathers(b)

    def outer(i, carry):
        for b in range(NBUF):
            j = i * NBUF + b
            r = row0 + j * CROWS
            rn = row0 + (j + NBUF) * CROWS
            wait_gathers(b)
            pltpu.async_copy(rows_v.at[b], out_hbm.at[pl.ds(r, CROWS)],
                             ssem.at[b])
            pltpu.async_copy(x_hbm.at[pl.ds(rn, CROWS)], idx_v.at[b],
                             isem.at[b])
            pltpu.make_async_copy(rows_v.at[b], out_hbm.at[pl.ds(r, CROWS)],
                                  ssem.at[b]).wait()
            pltpu.make_async_copy(x_hbm.at[pl.ds(rn, CROWS)], idx_v.at[b],
                                  isem.at[b]).wait()
            fire_gathers(b)
        return carry

    lax.fori_loop(0, NCHUNK // NBUF - 1, outer, 0)

    for b in range(NBUF):
        r = row0 + (NCHUNK - NBUF + b) * CROWS
        wait_gathers(b)
        pltpu.async_copy(rows_v.at[b], out_hbm.at[pl.ds(r, CROWS)],
                         ssem.at[b])
    for b in range(NBUF):
        r = row0 + (NCHUNK - NBUF + b) * CROWS
        pltpu.make_async_copy(rows_v.at[b], out_hbm.at[pl.ds(r, CROWS)],
                              ssem.at[b]).wait()


def kernel(x, weight):
    return _embed(x.astype(jnp.int32), weight)
